# baseline (device time: 25174 ns/iter reference)
import jax
import jax.numpy as jnp
from jax import lax
from jax.experimental import pallas as pl
from jax.experimental.pallas import tpu as pltpu

N_DEV = 16


def kernel(x, w_mat):
    k, m_per = x.shape
    n = w_mat.shape[1]
    blk = k // N_DEV
    assert blk == m_per, (blk, m_per)

    def body(x_ref, w_ref, out_ref, gather_ref, send_sems, recv_sems):
        me = lax.axis_index("i")

        barrier = pltpu.get_barrier_semaphore()
        for off in range(1, N_DEV):
            peer = lax.rem(me + off, N_DEV)
            pl.semaphore_signal(
                barrier, inc=1, device_id=(peer,),
                device_id_type=pl.DeviceIdType.MESH,
            )
        pl.semaphore_wait(barrier, N_DEV - 1)

        rdmas = []
        for off in range(1, N_DEV):
            dst = lax.rem(me + off, N_DEV)
            rdma = pltpu.make_async_remote_copy(
                src_ref=x_ref.at[pl.ds(dst * blk, blk)],
                dst_ref=gather_ref.at[off],
                send_sem=send_sems.at[off],
                recv_sem=recv_sems.at[off],
                device_id=(dst,),
                device_id_type=pl.DeviceIdType.MESH,
            )
            rdma.start()
            rdmas.append(rdma)

        out_ref[...] = jnp.dot(
            x_ref[pl.ds(me * blk, blk), :],
            w_ref[pl.ds(me * blk, blk), :],
            preferred_element_type=jnp.float32,
        )

        for off in range(1, N_DEV):
            src = lax.rem(me - off + N_DEV, N_DEV)
            rdmas[off - 1].wait_recv()
            out_ref[...] += jnp.dot(
                gather_ref[off],
                w_ref[pl.ds(src * blk, blk), :],
                preferred_element_type=jnp.float32,
            )

        for off in range(1, N_DEV):
            rdmas[off - 1].wait_send()

    try:
        params = pltpu.CompilerParams(collective_id=0)
    except AttributeError:
        params = pltpu.TPUCompilerParams(collective_id=0)

    return pl.pallas_call(
        body,
        out_shape=jax.ShapeDtypeStruct((blk, n), jnp.float32),
        in_specs=[
            pl.BlockSpec(memory_space=pltpu.VMEM),
            pl.BlockSpec(memory_space=pltpu.VMEM),
        ],
        out_specs=pl.BlockSpec(memory_space=pltpu.VMEM),
        scratch_shapes=[
            pltpu.VMEM((N_DEV, blk, blk), x.dtype),
            pltpu.SemaphoreType.DMA((N_DEV,)),
            pltpu.SemaphoreType.DMA((N_DEV,)),
        ],
        compiler_params=params,
    )(x, w_mat)


# device time: 21655 ns/iter; 1.1625x vs baseline; 1.1625x over previous
import jax
import jax.numpy as jnp
from jax import lax
from jax.experimental import pallas as pl
from jax.experimental.pallas import tpu as pltpu

N_DEV = 16


def kernel(x, w_mat):
    k, m_per = x.shape
    n = w_mat.shape[1]
    blk = k // N_DEV
    assert blk == m_per, (blk, m_per)

    def body(x_ref, w_ref, out_ref, xb_ref, wb_ref, gather_ref,
             send_sems, recv_sems):
        me = lax.axis_index("i")

        xb_ref[...] = x_ref[...].astype(jnp.bfloat16)

        barrier = pltpu.get_barrier_semaphore()
        for off in range(1, N_DEV):
            peer = lax.rem(me + off, N_DEV)
            pl.semaphore_signal(
                barrier, inc=1, device_id=(peer,),
                device_id_type=pl.DeviceIdType.MESH,
            )
        pl.semaphore_wait(barrier, N_DEV - 1)

        rdmas = []
        for off in range(1, N_DEV):
            dst = lax.rem(me + off, N_DEV)
            rdma = pltpu.make_async_remote_copy(
                src_ref=xb_ref.at[pl.ds(dst * blk, blk)],
                dst_ref=gather_ref.at[off],
                send_sem=send_sems.at[off],
                recv_sem=recv_sems.at[off],
                device_id=(dst,),
                device_id_type=pl.DeviceIdType.MESH,
            )
            rdma.start()
            rdmas.append(rdma)

        wb_ref[...] = w_ref[...].astype(jnp.bfloat16)

        out_ref[...] = jnp.dot(
            xb_ref[pl.ds(me * blk, blk), :],
            wb_ref[pl.ds(me * blk, blk), :],
            preferred_element_type=jnp.float32,
        )

        for off in range(1, N_DEV):
            src = lax.rem(me - off + N_DEV, N_DEV)
            rdmas[off - 1].wait_recv()
            out_ref[...] += jnp.dot(
                gather_ref[off],
                wb_ref[pl.ds(src * blk, blk), :],
                preferred_element_type=jnp.float32,
            )

        for off in range(1, N_DEV):
            rdmas[off - 1].wait_send()

    try:
        params = pltpu.CompilerParams(collective_id=0)
    except AttributeError:
        params = pltpu.TPUCompilerParams(collective_id=0)

    return pl.pallas_call(
        body,
        out_shape=jax.ShapeDtypeStruct((blk, n), jnp.float32),
        in_specs=[
            pl.BlockSpec(memory_space=pltpu.VMEM),
            pl.BlockSpec(memory_space=pltpu.VMEM),
        ],
        out_specs=pl.BlockSpec(memory_space=pltpu.VMEM),
        scratch_shapes=[
            pltpu.VMEM((k, m_per), jnp.bfloat16),
            pltpu.VMEM((k, n), jnp.bfloat16),
            pltpu.VMEM((N_DEV, blk, blk), jnp.bfloat16),
            pltpu.SemaphoreType.DMA((N_DEV,)),
            pltpu.SemaphoreType.DMA((N_DEV,)),
        ],
        compiler_params=params,
    )(x, w_mat)


# device time: 20999 ns/iter; 1.1988x vs baseline; 1.0312x over previous
import jax
import jax.numpy as jnp
from jax import lax
from jax.experimental import pallas as pl
from jax.experimental.pallas import tpu as pltpu

N_DEV = 16


def kernel(x, w_mat):
    k, m_per = x.shape
    n = w_mat.shape[1]
    blk = k // N_DEV
    assert blk == m_per, (blk, m_per)

    def body(x_ref, w_ref, out_ref, xb_ref, wb_ref, gather_ref,
             send_sems, recv_sems, local_sem):
        me = lax.axis_index("i")

        xb_ref[...] = x_ref[...].astype(jnp.bfloat16)

        barrier = pltpu.get_barrier_semaphore()
        for off in range(1, N_DEV):
            peer = lax.rem(me + off, N_DEV)
            pl.semaphore_signal(
                barrier, inc=1, device_id=(peer,),
                device_id_type=pl.DeviceIdType.MESH,
            )
        pl.semaphore_wait(barrier, N_DEV - 1)

        rdmas = []
        for off in range(1, N_DEV):
            dst = lax.rem(me + off, N_DEV)
            rdma = pltpu.make_async_remote_copy(
                src_ref=xb_ref.at[pl.ds(dst * blk, blk)],
                dst_ref=gather_ref.at[me],
                send_sem=send_sems.at[off],
                recv_sem=recv_sems.at[me],
                device_id=(dst,),
                device_id_type=pl.DeviceIdType.MESH,
            )
            rdma.start()
            rdmas.append(rdma)

        local_cp = pltpu.make_async_copy(
            xb_ref.at[pl.ds(me * blk, blk)], gather_ref.at[me], local_sem,
        )
        local_cp.start()

        wb_ref[...] = w_ref[...].astype(jnp.bfloat16)

        out_ref[...] = jnp.zeros((blk, n), jnp.float32)

        for s in range(N_DEV):
            @pl.when(s == me)
            def _():
                local_cp.wait()

            @pl.when(s != me)
            def _():
                recv = pltpu.make_async_remote_copy(
                    src_ref=xb_ref.at[pl.ds(0, blk)],
                    dst_ref=gather_ref.at[s],
                    send_sem=send_sems.at[0],
                    recv_sem=recv_sems.at[s],
                    device_id=(0,),
                    device_id_type=pl.DeviceIdType.MESH,
                )
                recv.wait_recv()

            out_ref[...] += jnp.dot(
                gather_ref[s],
                wb_ref[s * blk:(s + 1) * blk, :],
                preferred_element_type=jnp.float32,
            )

        for off in range(1, N_DEV):
            rdmas[off - 1].wait_send()

    try:
        params = pltpu.CompilerParams(collective_id=0)
    except AttributeError:
        params = pltpu.TPUCompilerParams(collective_id=0)

    return pl.pallas_call(
        body,
        out_shape=jax.ShapeDtypeStruct((blk, n), jnp.float32),
        in_specs=[
            pl.BlockSpec(memory_space=pltpu.VMEM),
            pl.BlockSpec(memory_space=pltpu.VMEM),
        ],
        out_specs=pl.BlockSpec(memory_space=pltpu.VMEM),
        scratch_shapes=[
            pltpu.VMEM((k, m_per), jnp.bfloat16),
            pltpu.VMEM((k, n), jnp.bfloat16),
            pltpu.VMEM((N_DEV, blk, blk), jnp.bfloat16),
            pltpu.SemaphoreType.DMA((N_DEV,)),
            pltpu.SemaphoreType.DMA((N_DEV,)),
            pltpu.SemaphoreType.DMA,
        ],
        compiler_params=params,
    )(x, w_mat)
